# speedup vs baseline: 1.8731x; 1.8731x over previous
import functools

import jax
import jax.numpy as jnp
from jax import lax
from jax.experimental import pallas as pl
from jax.experimental.pallas import tpu as pltpu
from jax.experimental.pallas import tpu_sc as plsc

_VOCAB = 100000
_EMB = 1024
_HID = 2048
_OUT = 128
_BATCH = 4096

_NC = 2
_NS = 16
_NW = _NC * _NS
_BPW = _BATCH // _NW
_CHUNK = 32
_NCHUNK = _BPW // _CHUNK


def _sc_gather(idx3, table):
    mesh = plsc.VectorSubcoreMesh(core_axis_name="c", subcore_axis_name="s")

    @functools.partial(
        pl.kernel,
        mesh=mesh,
        out_type=jax.ShapeDtypeStruct((_BATCH, _EMB), jnp.float32),
        scratch_types=[
            pltpu.VMEM((_NCHUNK, _CHUNK), jnp.int32),
            pltpu.VMEM((_CHUNK, _EMB), jnp.float32),
            pltpu.VMEM((_CHUNK, _EMB), jnp.float32),
            pltpu.SemaphoreType.DMA,
            pltpu.SemaphoreType.DMA,
        ],
    )
    def gather_kernel(idx_hbm, table_hbm, out_hbm, idx_v, buf0, buf1, sem0, sem1):
        wid = lax.axis_index("s") * _NC + lax.axis_index("c")
        base = wid * _BPW
        pltpu.sync_copy(idx_hbm.at[wid], idx_v)
        bufs = (buf0, buf1)
        sems = (sem0, sem1)
        cps = [None] * _NCHUNK
        for k in range(_NCHUNK):
            if k >= 2:
                cps[k - 2].wait()
                pltpu.sync_copy(bufs[(k - 2) % 2],
                                out_hbm.at[pl.ds(base + (k - 2) * _CHUNK, _CHUNK)])
            cps[k] = pltpu.async_copy(table_hbm.at[idx_v.at[k]], bufs[k % 2],
                                      sems[k % 2])
        for k in range(_NCHUNK - 2, _NCHUNK):
            cps[k].wait()
            pltpu.sync_copy(bufs[k % 2],
                            out_hbm.at[pl.ds(base + k * _CHUNK, _CHUNK)])

    return gather_kernel(idx3, table)


_BM = 512


def _mlp_body(emb_ref, w1_ref, b1_ref, w2_ref, b2_ref, out_ref):
    h = jnp.dot(emb_ref[...], w1_ref[...], preferred_element_type=jnp.float32)
    h = jnp.maximum(h + b1_ref[...], 0.0)
    out_ref[...] = (
        jnp.dot(h, w2_ref[...], preferred_element_type=jnp.float32) + b2_ref[...]
    )


def _tc_mlp(emb, W1, b1, W2, b2):
    grid = (_BATCH // _BM,)
    return pl.pallas_call(
        _mlp_body,
        grid=grid,
        in_specs=[
            pl.BlockSpec((_BM, _EMB), lambda i: (i, 0)),
            pl.BlockSpec((_EMB, _HID), lambda i: (0, 0)),
            pl.BlockSpec((1, _HID), lambda i: (0, 0)),
            pl.BlockSpec((_HID, _OUT), lambda i: (0, 0)),
            pl.BlockSpec((1, _OUT), lambda i: (0, 0)),
        ],
        out_specs=pl.BlockSpec((_BM, _OUT), lambda i: (i, 0)),
        out_shape=jax.ShapeDtypeStruct((_BATCH, _OUT), jnp.float32),
        compiler_params=pltpu.CompilerParams(
            dimension_semantics=("arbitrary",),
        ),
    )(emb, W1, b1, W2, b2)


def kernel(user_one_hot_vector, table, W1, b1, W2, b2):
    idx3 = user_one_hot_vector.astype(jnp.int32).reshape(_NW, _NCHUNK, _CHUNK)
    emb = _sc_gather(idx3, table)
    return _tc_mlp(emb, W1, b1.reshape(1, _HID), W2, b2.reshape(1, _OUT))
